# baseline (speedup 1.0000x reference)
import functools

import jax
import jax.numpy as jnp
import numpy as np
from jax.experimental import pallas as pl

H = 8
F32 = jnp.float32
BF16 = jnp.bfloat16


def _dot(a, b):
    return jnp.dot(a, b, preferred_element_type=F32)


def _bdot(a, b):
    return jnp.dot(a.astype(BF16), b, preferred_element_type=F32)


def _edge_body(xi_ref, xj_ref, eg_ref,
               We1a_ref, We1b_ref, We1c_ref, be1_ref, We2_ref, be2_ref,
               Wca_ref, Wcb_ref, bc_ref, A1_ref, ba1_ref, A2_ref, ba2_ref,
               Wv_ref, bv_ref, P_ref,
               trip_ref, prob_ref, val_ref):
    xi = xi_ref[...].astype(BF16)
    xj = xj_ref[...].astype(BF16)
    eg = eg_ref[...].astype(BF16)
    pre = (_dot(xi, We1a_ref[...]) + _dot(eg, We1b_ref[...])
           + _dot(xj, We1c_ref[...]) + be1_ref[...])
    trip_ref[...] = _bdot(jax.nn.relu(pre), We2_ref[...]) + be2_ref[...]

    c = _dot(xi, Wca_ref[...]) + _dot(eg, Wcb_ref[...]) + bc_ref[...]
    hh = jax.nn.relu(_bdot(c, A1_ref[...]) + ba1_ref[...])
    att = (_bdot(hh, A2_ref[...]) + ba2_ref[...]) * (1.0 / np.sqrt(32.0))
    probs = []
    for h in range(H):
        ah = att[:, 32 * h:32 * h + 32]
        m = jnp.max(ah, axis=1, keepdims=True)
        e = jnp.exp(ah - m)
        probs.append(e / jnp.sum(e, axis=1, keepdims=True))
    prob_hm = jnp.concatenate(probs, axis=1)
    prob_flat = _bdot(prob_hm, P_ref[...])
    prob_ref[...] = prob_flat
    v = _dot(xj, Wv_ref[...]) + bv_ref[...]
    val_ref[...] = prob_flat * v


def _edge_call(xi, xj, eg, We1a, We1b, We1c, be1, We2, be2,
               Wca, Wcb, bc, A1, ba1, A2, ba2, Wv, bv, P, BE):
    E = xi.shape[0]
    DN = xi.shape[1]
    grid = E // BE
    row = lambda i: (i, 0)
    full = lambda i: (0, 0)
    bspec_e = pl.BlockSpec((BE, DN), row)
    wspec = lambda a: pl.BlockSpec(a.shape, full)
    return pl.pallas_call(
        _edge_body,
        interpret=False,
        grid=(grid,),
        in_specs=[bspec_e, bspec_e, bspec_e] + [wspec(a) for a in (
            We1a, We1b, We1c, be1, We2, be2, Wca, Wcb, bc, A1, ba1, A2, ba2,
            Wv, bv, P)],
        out_specs=[bspec_e, bspec_e, bspec_e],
        out_shape=[jax.ShapeDtypeStruct((E, DN), F32),
                   jax.ShapeDtypeStruct((E, DN), F32),
                   jax.ShapeDtypeStruct((E, DN), F32)],
    )(xi, xj, eg, We1a, We1b, We1c, be1, We2, be2,
      Wca, Wcb, bc, A1, ba1, A2, ba2, Wv, bv, P)


def _mv_body(xi2_ref, ids0_ref, image_ref,
             Wq2_ref, bq2_ref, Wk2_ref, bk2_ref, Wv2_ref, bv2_ref, y_ref):
    E2 = xi2_ref.shape[0]
    M = image_ref.shape[0]
    ids0 = ids0_ref[...]
    iota = jax.lax.broadcasted_iota(jnp.int32, (E2, M), 1)
    oh = (iota == ids0).astype(F32)
    xj2 = _dot(oh, image_ref[...])
    q2 = _dot(xi2_ref[...], Wq2_ref[...]) + bq2_ref[...]
    k2 = _dot(xj2, Wk2_ref[...]) + bk2_ref[...]
    v2 = _dot(xj2, Wv2_ref[...]) + bv2_ref[...]
    scale = 1.0 / np.sqrt(256.0)
    ys = []
    for h in range(H):
        qh = q2[:, 32 * h:32 * h + 32]
        kh = k2[:, 32 * h:32 * h + 32]
        vh = v2[:, 32 * h:32 * h + 32]
        s = jax.lax.dot_general(qh, kh, (((1,), (1,)), ((), ())),
                                preferred_element_type=F32) * scale
        m = jnp.max(s, axis=1, keepdims=True)
        e = jnp.exp(s - m)
        a = e / jnp.sum(e, axis=1, keepdims=True)
        ys.append(_dot(a, vh))
    y_ref[...] = jnp.concatenate(ys, axis=1)


def _mv_call(xi2, ids0, image, Wq2, bq2, Wk2, bk2, Wv2, bv2):
    E2, DN = xi2.shape
    args = (xi2, ids0, image, Wq2, bq2, Wk2, bk2, Wv2, bv2)
    return pl.pallas_call(
        _mv_body,
        interpret=False,
        in_specs=[pl.BlockSpec(a.shape, lambda: (0,) * 2) for a in args],
        out_specs=pl.BlockSpec((E2, DN), lambda: (0, 0)),
        out_shape=jax.ShapeDtypeStruct((E2, DN), F32),
    )(*args)


def _final_body(node_ref, agg_ref, y_ref, ids1_ref,
                Wu1a_ref, Wu1b_ref, bu1_ref, Wu2_ref, bu2_ref,
                Wnna_ref, Wnnb_ref, bnn_ref, out_ref, *, BN):
    i = pl.program_id(0)
    E2 = y_ref.shape[0]
    nf = jax.nn.relu(_dot(node_ref[...], Wu1a_ref[...])
                     + _dot(agg_ref[...], Wu1b_ref[...]) + bu1_ref[...])
    node_fan = _dot(nf, Wu2_ref[...]) + bu2_ref[...]
    rowids = jax.lax.broadcasted_iota(jnp.int32, (BN, E2), 0) + i * BN
    oh = (rowids == ids1_ref[...]).astype(F32)
    img = _dot(oh, y_ref[...])
    out_ref[...] = (_dot(node_fan, Wnna_ref[...]) + _dot(img, Wnnb_ref[...])
                    + bnn_ref[...])


def _final_call(node, agg, y, ids1, Wu1a, Wu1b, bu1, Wu2, bu2,
                Wnna, Wnnb, bnn, BN):
    N, DN = node.shape
    grid = N // BN
    row = lambda i: (i, 0)
    full = lambda i: (0, 0)
    nspec = pl.BlockSpec((BN, DN), row)
    args = (node, agg, y, ids1, Wu1a, Wu1b, bu1, Wu2, bu2, Wnna, Wnnb, bnn)
    return pl.pallas_call(
        functools.partial(_final_body, BN=BN),
        interpret=False,
        grid=(grid,),
        in_specs=[nspec, nspec] + [pl.BlockSpec(a.shape, full)
                                   for a in args[2:]],
        out_specs=nspec,
        out_shape=jax.ShapeDtypeStruct((N, DN), F32),
    )(*args)


def kernel(node, image, edge, edge_index_node_2_node, edge_index_image_2_ndoe,
           Wq, bq, Wk, bk, Wv, bv, We1, be1, We2, be2,
           Wa1, ba1, Wa2, ba2, Wu1, bu1, Wu2, bu2,
           Wq2, bq2, Wk2, bk2, Wv2, bv2, Wnn, bnn):
    N, DN = node.shape
    E = edge.shape[0]
    E2 = edge_index_image_2_ndoe.shape[1]
    ei = edge_index_node_2_node
    ei2 = edge_index_image_2_ndoe

    ar = jnp.arange(DN)
    hm = (ar % 32) * 8 + (ar // 32)
    P = jax.nn.one_hot(hm, DN, dtype=F32)
    Wc = jnp.zeros((2 * DN, 2 * DN), F32)
    colq = (jnp.arange(8).repeat(32) * 64) + jnp.tile(jnp.arange(32), 8)
    colk = colq + 32
    Wc = Wc.at[:DN, colq].set(Wq[:, hm])
    Wc = Wc.at[DN:, colk].set(Wk[:, hm])
    bc = jnp.zeros((2 * DN,), F32).at[colq].set(bq[hm]).at[colk].set(bk[hm])
    A1 = jnp.zeros((2 * DN, 2 * DN), F32)
    A2 = jnp.zeros((2 * DN, DN), F32)
    for h in range(H):
        A1 = A1.at[64 * h:64 * h + 64, 64 * h:64 * h + 64].set(Wa1.T)
        A2 = A2.at[64 * h:64 * h + 64, 32 * h:32 * h + 32].set(Wa2.T)
    ba1big = jnp.tile(ba1, H)
    ba2big = jnp.tile(ba2, H)
    We1a, We1b, We1c = We1[:DN], We1[DN:2 * DN], We1[2 * DN:]
    Wu1a, Wu1b = Wu1[:DN], Wu1[DN:]
    Wnna, Wnnb = Wnn[:DN], Wnn[DN:]
    r2 = lambda b: b.reshape(1, -1).astype(F32)
    b16 = lambda w: w.astype(BF16)

    x_i = jnp.take(node, ei[0], axis=0)
    x_j = jnp.take(node, ei[1], axis=0)
    xi2 = jnp.take(node, ei2[1], axis=0)

    BE = 640 if E % 640 == 0 else E
    trip, prob_flat, value = _edge_call(
        x_i, x_j, edge,
        b16(We1a), b16(We1b), b16(We1c), r2(be1), b16(We2), r2(be2),
        b16(Wc[:DN]), b16(Wc[DN:]), r2(bc),
        b16(A1), r2(ba1big), b16(A2), r2(ba2big),
        b16(Wv), r2(bv), b16(P), BE)

    agg = jax.ops.segment_sum(value, ei[0], num_segments=N)

    y = _mv_call(xi2, ei2[0].reshape(E2, 1), image,
                 Wq2, r2(bq2), Wk2, r2(bk2), Wv2, r2(bv2))

    node_update = _final_call(node, agg, y, ei2[1].reshape(1, E2),
                              Wu1a, Wu1b, r2(bu1), Wu2, r2(bu2),
                              Wnna, Wnnb, r2(bnn), BN=1000 if N % 1000 == 0 else N)

    return (node_update, trip, prob_flat.reshape(E, 32, 8))


# speedup vs baseline: 1.7524x; 1.7524x over previous
import functools

import jax
import jax.numpy as jnp
import numpy as np
from jax.experimental import pallas as pl

H = 8
F32 = jnp.float32
BF16 = jnp.bfloat16


def _dot(a, b):
    return jnp.dot(a, b, preferred_element_type=F32)


def _bdot(a, b):
    return jnp.dot(a.astype(BF16), b, preferred_element_type=F32)


def _edge_body(xi_ref, xj_ref, eg_ref,
               We1a_ref, We1b_ref, We1c_ref, be1_ref, We2_ref, be2_ref,
               Wca_ref, Wcb_ref, bc_ref, A1_ref, ba1_ref, A2_ref, ba2_ref,
               Wv_ref, bv_ref, P_ref,
               trip_ref, prob_ref, val_ref):
    xi = xi_ref[...].astype(BF16)
    xj = xj_ref[...].astype(BF16)
    eg = eg_ref[...].astype(BF16)
    pre = (_dot(xi, We1a_ref[...]) + _dot(eg, We1b_ref[...])
           + _dot(xj, We1c_ref[...]) + be1_ref[...])
    trip_ref[...] = _bdot(jax.nn.relu(pre), We2_ref[...]) + be2_ref[...]

    c = _dot(xi, Wca_ref[...]) + _dot(eg, Wcb_ref[...]) + bc_ref[...]
    hh = jax.nn.relu(_bdot(c, A1_ref[...]) + ba1_ref[...])
    att = (_bdot(hh, A2_ref[...]) + ba2_ref[...]) * (1.0 / np.sqrt(32.0))
    probs = []
    for h in range(H):
        ah = att[:, 32 * h:32 * h + 32]
        m = jnp.max(ah, axis=1, keepdims=True)
        e = jnp.exp(ah - m)
        probs.append(e / jnp.sum(e, axis=1, keepdims=True))
    prob_hm = jnp.concatenate(probs, axis=1)
    prob_flat = _bdot(prob_hm, P_ref[...])
    prob_ref[...] = prob_flat
    v = _dot(xj, Wv_ref[...]) + bv_ref[...]
    val_ref[...] = prob_flat * v


def _edge_call(xi, xj, eg, We1a, We1b, We1c, be1, We2, be2,
               Wca, Wcb, bc, A1, ba1, A2, ba2, Wv, bv, P, BE):
    E = xi.shape[0]
    DN = xi.shape[1]
    grid = E // BE
    row = lambda i: (i, 0)
    full = lambda i: (0, 0)
    bspec_e = pl.BlockSpec((BE, DN), row)
    wspec = lambda a: pl.BlockSpec(a.shape, full)
    return pl.pallas_call(
        _edge_body,
        interpret=False,
        grid=(grid,),
        in_specs=[bspec_e, bspec_e, bspec_e] + [wspec(a) for a in (
            We1a, We1b, We1c, be1, We2, be2, Wca, Wcb, bc, A1, ba1, A2, ba2,
            Wv, bv, P)],
        out_specs=[bspec_e, bspec_e, bspec_e],
        out_shape=[jax.ShapeDtypeStruct((E, DN), F32),
                   jax.ShapeDtypeStruct((E, DN), F32),
                   jax.ShapeDtypeStruct((E, DN), F32)],
    )(xi, xj, eg, We1a, We1b, We1c, be1, We2, be2,
      Wca, Wcb, bc, A1, ba1, A2, ba2, Wv, bv, P)


def _mv_body(xi2_ref, ids0_ref, image_ref,
             Wq2_ref, bq2_ref, Wk2_ref, bk2_ref, Wv2_ref, bv2_ref, y_ref):
    E2 = xi2_ref.shape[0]
    M = image_ref.shape[0]
    ids0 = ids0_ref[...]
    iota = jax.lax.broadcasted_iota(jnp.int32, (E2, M), 1)
    oh = (iota == ids0).astype(F32)
    xj2 = _dot(oh, image_ref[...])
    q2 = _dot(xi2_ref[...], Wq2_ref[...]) + bq2_ref[...]
    k2 = _dot(xj2, Wk2_ref[...]) + bk2_ref[...]
    v2 = _dot(xj2, Wv2_ref[...]) + bv2_ref[...]
    scale = 1.0 / np.sqrt(256.0)
    ys = []
    for h in range(H):
        qh = q2[:, 32 * h:32 * h + 32]
        kh = k2[:, 32 * h:32 * h + 32]
        vh = v2[:, 32 * h:32 * h + 32]
        s = jax.lax.dot_general(qh, kh, (((1,), (1,)), ((), ())),
                                preferred_element_type=F32) * scale
        m = jnp.max(s, axis=1, keepdims=True)
        e = jnp.exp(s - m)
        a = e / jnp.sum(e, axis=1, keepdims=True)
        ys.append(_dot(a, vh))
    y_ref[...] = jnp.concatenate(ys, axis=1)


def _mv_call(xi2, ids0, image, Wq2, bq2, Wk2, bk2, Wv2, bv2):
    E2, DN = xi2.shape
    args = (xi2, ids0, image, Wq2, bq2, Wk2, bk2, Wv2, bv2)
    return pl.pallas_call(
        _mv_body,
        interpret=False,
        in_specs=[pl.BlockSpec(a.shape, lambda: (0,) * 2) for a in args],
        out_specs=pl.BlockSpec((E2, DN), lambda: (0, 0)),
        out_shape=jax.ShapeDtypeStruct((E2, DN), F32),
    )(*args)


def _final_body(node_ref, agg_ref, y_ref, ids1_ref,
                Wu1a_ref, Wu1b_ref, bu1_ref, Wu2_ref, bu2_ref,
                Wnna_ref, Wnnb_ref, bnn_ref, out_ref, *, BN):
    i = pl.program_id(0)
    E2 = y_ref.shape[0]
    nf = jax.nn.relu(_dot(node_ref[...], Wu1a_ref[...])
                     + _dot(agg_ref[...], Wu1b_ref[...]) + bu1_ref[...])
    node_fan = _dot(nf, Wu2_ref[...]) + bu2_ref[...]
    rowids = jax.lax.broadcasted_iota(jnp.int32, (BN, E2), 0) + i * BN
    oh = (rowids == ids1_ref[...]).astype(F32)
    img = _dot(oh, y_ref[...])
    out_ref[...] = (_dot(node_fan, Wnna_ref[...]) + _dot(img, Wnnb_ref[...])
                    + bnn_ref[...])


def _final_call(node, agg, y, ids1, Wu1a, Wu1b, bu1, Wu2, bu2,
                Wnna, Wnnb, bnn, BN):
    N, DN = node.shape
    grid = N // BN
    row = lambda i: (i, 0)
    full = lambda i: (0, 0)
    nspec = pl.BlockSpec((BN, DN), row)
    args = (node, agg, y, ids1, Wu1a, Wu1b, bu1, Wu2, bu2, Wnna, Wnnb, bnn)
    return pl.pallas_call(
        functools.partial(_final_body, BN=BN),
        interpret=False,
        grid=(grid,),
        in_specs=[nspec, nspec] + [pl.BlockSpec(a.shape, full)
                                   for a in args[2:]],
        out_specs=nspec,
        out_shape=jax.ShapeDtypeStruct((N, DN), F32),
    )(*args)


def kernel(node, image, edge, edge_index_node_2_node, edge_index_image_2_ndoe,
           Wq, bq, Wk, bk, Wv, bv, We1, be1, We2, be2,
           Wa1, ba1, Wa2, ba2, Wu1, bu1, Wu2, bu2,
           Wq2, bq2, Wk2, bk2, Wv2, bv2, Wnn, bnn):
    N, DN = node.shape
    E = edge.shape[0]
    E2 = edge_index_image_2_ndoe.shape[1]
    ei = edge_index_node_2_node
    ei2 = edge_index_image_2_ndoe

    DH = DN // H
    to_hm = lambda W: W.reshape(-1, DH, H).transpose(0, 2, 1).reshape(-1, DN)
    Q_hm = to_hm(Wq)
    K_hm = to_hm(Wk)
    bq_hm = bq.reshape(DH, H).T.reshape(DN)
    bk_hm = bk.reshape(DH, H).T.reshape(DN)
    eyeN = jnp.eye(DN, dtype=F32)
    P = eyeN.reshape(DN, DH, H).transpose(0, 2, 1).reshape(DN, DN).T
    z = jnp.zeros((DN, H, DH), F32)
    Wca = jnp.concatenate([Q_hm.reshape(DN, H, DH), z], axis=2).reshape(DN, 2 * DN)
    Wcb = jnp.concatenate([z, K_hm.reshape(DN, H, DH)], axis=2).reshape(DN, 2 * DN)
    bc = jnp.concatenate([bq_hm.reshape(H, DH), bk_hm.reshape(H, DH)],
                         axis=1).reshape(2 * DN)
    A1 = jnp.kron(jnp.eye(H, dtype=F32), Wa1.T)
    A2 = jnp.kron(jnp.eye(H, dtype=F32), Wa2.T)
    ba1big = jnp.tile(ba1, H)
    ba2big = jnp.tile(ba2, H)
    We1a, We1b, We1c = We1[:DN], We1[DN:2 * DN], We1[2 * DN:]
    Wu1a, Wu1b = Wu1[:DN], Wu1[DN:]
    Wnna, Wnnb = Wnn[:DN], Wnn[DN:]
    r2 = lambda b: b.reshape(1, -1).astype(F32)
    b16 = lambda w: w.astype(BF16)

    x_i = jnp.take(node, ei[0], axis=0)
    x_j = jnp.take(node, ei[1], axis=0)
    xi2 = jnp.take(node, ei2[1], axis=0)

    BE = 640 if E % 640 == 0 else E
    trip, prob_flat, value = _edge_call(
        x_i, x_j, edge,
        b16(We1a), b16(We1b), b16(We1c), r2(be1), b16(We2), r2(be2),
        b16(Wca), b16(Wcb), r2(bc),
        b16(A1), r2(ba1big), b16(A2), r2(ba2big),
        b16(Wv), r2(bv), b16(P), BE)

    agg = jax.ops.segment_sum(value, ei[0], num_segments=N)

    y = _mv_call(xi2, ei2[0].reshape(E2, 1), image,
                 Wq2, r2(bq2), Wk2, r2(bk2), Wv2, r2(bv2))

    node_update = _final_call(node, agg, y, ei2[1].reshape(1, E2),
                              Wu1a, Wu1b, r2(bu1), Wu2, r2(bu2),
                              Wnna, Wnnb, r2(bnn), BN=1000 if N % 1000 == 0 else N)

    return (node_update, trip, prob_flat.reshape(E, 32, 8))


# speedup vs baseline: 2.3452x; 1.3383x over previous
import functools

import jax
import jax.numpy as jnp
import numpy as np
from jax.experimental import pallas as pl

H = 8
F32 = jnp.float32
BF16 = jnp.bfloat16


def _dot(a, b):
    return jnp.dot(a, b, preferred_element_type=F32)


def _bdot(a, b):
    return jnp.dot(a.astype(BF16), b, preferred_element_type=F32)


def _edge_body(xi_ref, xj_ref, eg_ref,
               We1a_ref, We1b_ref, We1c_ref, be1_ref, We2_ref, be2_ref,
               Wca_ref, Wcb_ref, bc_ref, A1_ref, ba1_ref, A2_ref, ba2_ref,
               Wv_ref, bv_ref, P_ref, GP_ref,
               trip_ref, prob_ref, val_ref):
    xi = xi_ref[...]
    xj = xj_ref[...]
    eg = eg_ref[...]
    pre = (_dot(xi, We1a_ref[...]) + _dot(eg, We1b_ref[...])
           + _dot(xj, We1c_ref[...]) + be1_ref[...])
    trip_ref[...] = _bdot(jax.nn.relu(pre), We2_ref[...]) + be2_ref[...]

    c = _dot(xi, Wca_ref[...]) + _dot(eg, Wcb_ref[...]) + bc_ref[...]
    hh = jax.nn.relu(_bdot(c, A1_ref[...]) + ba1_ref[...])
    att = (_bdot(hh, A2_ref[...]) + ba2_ref[...]) * (1.0 / np.sqrt(32.0))
    m = jnp.max(att, axis=1, keepdims=True)
    e = jnp.exp(att - m)
    e_flat = _bdot(e, P_ref[...])
    s_flat = _bdot(e, GP_ref[...])
    prob_flat = e_flat / s_flat
    prob_ref[...] = prob_flat
    v = _dot(xj, Wv_ref[...]) + bv_ref[...]
    val_ref[...] = prob_flat * v


def _edge_call(xi, xj, eg, We1a, We1b, We1c, be1, We2, be2,
               Wca, Wcb, bc, A1, ba1, A2, ba2, Wv, bv, P, GP, BE):
    E = xi.shape[0]
    DN = xi.shape[1]
    grid = E // BE
    row = lambda i: (i, 0)
    full = lambda i: (0, 0)
    bspec_e = pl.BlockSpec((BE, DN), row)
    wspec = lambda a: pl.BlockSpec(a.shape, full)
    return pl.pallas_call(
        _edge_body,
        interpret=False,
        grid=(grid,),
        in_specs=[bspec_e, bspec_e, bspec_e] + [wspec(a) for a in (
            We1a, We1b, We1c, be1, We2, be2, Wca, Wcb, bc, A1, ba1, A2, ba2,
            Wv, bv, P, GP)],
        out_specs=[bspec_e, bspec_e, bspec_e],
        out_shape=[jax.ShapeDtypeStruct((E, DN), F32),
                   jax.ShapeDtypeStruct((E, DN), F32),
                   jax.ShapeDtypeStruct((E, DN), F32)],
    )(xi, xj, eg, We1a, We1b, We1c, be1, We2, be2,
      Wca, Wcb, bc, A1, ba1, A2, ba2, Wv, bv, P, GP)


def _mv_body(xi2_ref, ids0_ref, image_ref,
             Wq2_ref, bq2_ref, Wk2_ref, bk2_ref, Wv2_ref, bv2_ref, y_ref):
    E2 = xi2_ref.shape[0]
    M = image_ref.shape[0]
    ids0 = ids0_ref[...]
    iota = jax.lax.broadcasted_iota(jnp.int32, (E2, M), 1)
    oh = (iota == ids0).astype(F32)
    xj2 = _dot(oh, image_ref[...])
    q2 = _dot(xi2_ref[...], Wq2_ref[...]) + bq2_ref[...]
    k2 = _dot(xj2, Wk2_ref[...]) + bk2_ref[...]
    v2 = _dot(xj2, Wv2_ref[...]) + bv2_ref[...]
    scale = 1.0 / np.sqrt(256.0)
    ys = []
    for h in range(H):
        qh = q2[:, 32 * h:32 * h + 32]
        kh = k2[:, 32 * h:32 * h + 32]
        vh = v2[:, 32 * h:32 * h + 32]
        s = jax.lax.dot_general(qh, kh, (((1,), (1,)), ((), ())),
                                preferred_element_type=F32) * scale
        m = jnp.max(s, axis=1, keepdims=True)
        e = jnp.exp(s - m)
        a = e / jnp.sum(e, axis=1, keepdims=True)
        ys.append(_dot(a, vh))
    y_ref[...] = jnp.concatenate(ys, axis=1)


def _mv_call(xi2, ids0, image, Wq2, bq2, Wk2, bk2, Wv2, bv2):
    E2, DN = xi2.shape
    args = (xi2, ids0, image, Wq2, bq2, Wk2, bk2, Wv2, bv2)
    return pl.pallas_call(
        _mv_body,
        interpret=False,
        in_specs=[pl.BlockSpec(a.shape, lambda: (0,) * 2) for a in args],
        out_specs=pl.BlockSpec((E2, DN), lambda: (0, 0)),
        out_shape=jax.ShapeDtypeStruct((E2, DN), F32),
    )(*args)


def _final_body(node_ref, agg_ref, y_ref, ids1_ref,
                Wu1a_ref, Wu1b_ref, bu1_ref, Wu2_ref, bu2_ref,
                Wnna_ref, Wnnb_ref, bnn_ref, out_ref, *, BN):
    i = pl.program_id(0)
    E2 = y_ref.shape[0]
    y16 = y_ref[...].astype(BF16)
    nf = jax.nn.relu(_bdot(node_ref[...], Wu1a_ref[...])
                     + _bdot(agg_ref[...], Wu1b_ref[...]) + bu1_ref[...])
    node_fan = _bdot(nf, Wu2_ref[...]) + bu2_ref[...]
    rowids = jax.lax.broadcasted_iota(jnp.int32, (BN, E2), 0) + i * BN
    oh = (rowids == ids1_ref[...]).astype(BF16)
    img = jnp.dot(oh, y16, preferred_element_type=F32)
    out_ref[...] = (_bdot(node_fan, Wnna_ref[...]) + _bdot(img, Wnnb_ref[...])
                    + bnn_ref[...])


def _final_call(node, agg, y, ids1, Wu1a, Wu1b, bu1, Wu2, bu2,
                Wnna, Wnnb, bnn, BN):
    N, DN = node.shape
    grid = N // BN
    row = lambda i: (i, 0)
    full = lambda i: (0, 0)
    nspec = pl.BlockSpec((BN, DN), row)
    args = (node, agg, y, ids1, Wu1a, Wu1b, bu1, Wu2, bu2, Wnna, Wnnb, bnn)
    return pl.pallas_call(
        functools.partial(_final_body, BN=BN),
        interpret=False,
        grid=(grid,),
        in_specs=[nspec, nspec] + [pl.BlockSpec(a.shape, full)
                                   for a in args[2:]],
        out_specs=nspec,
        out_shape=jax.ShapeDtypeStruct((N, DN), F32),
    )(*args)


def kernel(node, image, edge, edge_index_node_2_node, edge_index_image_2_ndoe,
           Wq, bq, Wk, bk, Wv, bv, We1, be1, We2, be2,
           Wa1, ba1, Wa2, ba2, Wu1, bu1, Wu2, bu2,
           Wq2, bq2, Wk2, bk2, Wv2, bv2, Wnn, bnn):
    N, DN = node.shape
    E = edge.shape[0]
    E2 = edge_index_image_2_ndoe.shape[1]
    ei = edge_index_node_2_node
    ei2 = edge_index_image_2_ndoe

    DH = DN // H
    to_hm = lambda W: W.reshape(-1, DH, H).transpose(0, 2, 1).reshape(-1, DN)
    Q_hm = to_hm(Wq)
    K_hm = to_hm(Wk)
    bq_hm = bq.reshape(DH, H).T.reshape(DN)
    bk_hm = bk.reshape(DH, H).T.reshape(DN)
    eyeN = jnp.eye(DN, dtype=F32)
    P = eyeN.reshape(DN, DH, H).transpose(0, 2, 1).reshape(DN, DN).T
    G = jnp.kron(jnp.eye(H, dtype=F32), jnp.ones((DH, DH), F32))
    GP = _dot(G, P)
    z = jnp.zeros((DN, H, DH), F32)
    Wca = jnp.concatenate([Q_hm.reshape(DN, H, DH), z], axis=2).reshape(DN, 2 * DN)
    Wcb = jnp.concatenate([z, K_hm.reshape(DN, H, DH)], axis=2).reshape(DN, 2 * DN)
    bc = jnp.concatenate([bq_hm.reshape(H, DH), bk_hm.reshape(H, DH)],
                         axis=1).reshape(2 * DN)
    A1 = jnp.kron(jnp.eye(H, dtype=F32), Wa1.T)
    A2 = jnp.kron(jnp.eye(H, dtype=F32), Wa2.T)
    ba1big = jnp.tile(ba1, H)
    ba2big = jnp.tile(ba2, H)
    We1a, We1b, We1c = We1[:DN], We1[DN:2 * DN], We1[2 * DN:]
    Wu1a, Wu1b = Wu1[:DN], Wu1[DN:]
    Wnna, Wnnb = Wnn[:DN], Wnn[DN:]
    r2 = lambda b: b.reshape(1, -1).astype(F32)
    b16 = lambda w: w.astype(BF16)

    node16 = node.astype(BF16)
    edge16 = edge.astype(BF16)
    x_i = jnp.take(node16, ei[0], axis=0)
    x_j = jnp.take(node16, ei[1], axis=0)
    xi2 = jnp.take(node, ei2[1], axis=0)

    BE = 640 if E % 640 == 0 else E
    trip, prob_flat, value = _edge_call(
        x_i, x_j, edge16,
        b16(We1a), b16(We1b), b16(We1c), r2(be1), b16(We2), r2(be2),
        b16(Wca), b16(Wcb), r2(bc),
        b16(A1), r2(ba1big), b16(A2), r2(ba2big),
        b16(Wv), r2(bv), b16(P), b16(GP), BE)

    agg = jax.ops.segment_sum(value, ei[0], num_segments=N)

    y = _mv_call(xi2, ei2[0].reshape(E2, 1), image,
                 Wq2, r2(bq2), Wk2, r2(bk2), Wv2, r2(bv2))

    node_update = _final_call(node16, agg, y, ei2[1].reshape(1, E2),
                              b16(Wu1a), b16(Wu1b), r2(bu1), b16(Wu2), r2(bu2),
                              b16(Wnna), b16(Wnnb), r2(bnn),
                              BN=1000 if N % 1000 == 0 else N)

    return (node_update, trip, prob_flat.reshape(E, 32, 8))
